# Initial kernel scaffold; baseline (speedup 1.0000x reference)
import jax
import jax.numpy as jnp
from jax.experimental import pallas as pl


def kernel(inputs):
    raise NotImplementedError("write your pallas kernel here")


# speedup vs baseline: 2.0260x; 2.0260x over previous
import functools

import jax
import jax.numpy as jnp
from jax import lax
from jax.experimental import pallas as pl
from jax.experimental.pallas import tpu as pltpu
from jax.experimental.pallas import tpu_sc as plsc

SENT = 1000
N_IMGS = 64000
IMGS_PER_TILE = N_IMGS // 32
GROUPS = IMGS_PER_TILE // 80


def _v(x):
    return jnp.full((16,), x, jnp.int32)


def _sc_stats_kernel(x_hbm, f_hbm, n_hbm, s_hbm, xbuf, lab, lab2, hist,
                     fbuf, nbuf, sbuf):
    wid = lax.axis_index("s") * 2 + lax.axis_index("c")
    l16 = lax.iota(jnp.int32, 16)
    sent_v = _v(SENT)
    ones_v = jnp.ones((16,), jnp.int32)
    zero_v = jnp.zeros((16,), jnp.int32)

    for r in list(range(10)) + list(range(110, 128)):
        lab[r, :] = sent_v
        lab2[r, :] = sent_v

    def group_body(g, _):
        base = wid * (IMGS_PER_TILE * 300) + g * 24000
        pltpu.sync_copy(x_hbm.at[pl.ds(base, 24000)], xbuf)

        def col_body(c, _):
            j = _v(c * 16) + l16
            cen_base = ((j // _v(10)) * _v(10) + _v(4)) * _v(300) + _v(282)
            cen0 = plsc.load_gather(xbuf, [cen_base])
            cen1 = plsc.load_gather(xbuf, [cen_base + ones_v])
            cen2 = plsc.load_gather(xbuf, [cen_base + _v(2)])
            xb = j * _v(300)

            def binz(p, _):
                i0 = xb + _v(p * 3)
                d0 = jnp.abs(plsc.load_gather(xbuf, [i0]) - cen0)
                d1 = jnp.abs(plsc.load_gather(xbuf, [i0 + ones_v]) - cen1)
                d2 = jnp.abs(plsc.load_gather(xbuf, [i0 + _v(2)]) - cen2)
                m = jnp.maximum(jnp.maximum(d0, d1), d2)
                fg = m <= jnp.full((16,), 10.0, jnp.float32)
                lab[p + 10, :] = jnp.where(fg, _v(p), sent_v)
                hist[p, :] = zero_v
                return 0

            lax.fori_loop(0, 100, binz, 0)

            def cc_cond(carry):
                return carry > 0

            def cc_body(carry):
                def pass1(p, _):
                    cur = lab[p + 10, :]
                    up = lab[p, :]
                    dn = lab[p + 20, :]
                    pv = _v(p)
                    lf = jnp.where(pv % _v(10) == zero_v, sent_v,
                                   lab[p + 9, :])
                    rt = jnp.where(pv % _v(10) == _v(9), sent_v,
                                   lab[p + 11, :])
                    m = jnp.minimum(jnp.minimum(cur, jnp.minimum(up, dn)),
                                    jnp.minimum(lf, rt))
                    lab2[p + 10, :] = jnp.where(cur != sent_v, m, sent_v)
                    return 0

                lax.fori_loop(0, 100, pass1, 0)

                def pass2(p, ch):
                    v = lab2[p + 10, :]
                    row = jnp.minimum(v, _v(100)) + _v(10)
                    gth = plsc.load_gather(lab2, [row, l16])
                    nv = jnp.where(v != sent_v, jnp.minimum(v, gth), sent_v)
                    old = lab[p + 10, :]
                    lab[p + 10, :] = nv
                    d = jnp.max((nv != old).astype(jnp.int32))
                    return jnp.maximum(ch, d)

                return lax.fori_loop(0, 100, pass2, 0)

            lax.while_loop(cc_cond, cc_body, jnp.int32(1))

            def stats(p, carry):
                fc, nc = carry
                v = lab[p + 10, :]
                fg = v != sent_v
                fc = fc + fg.astype(jnp.int32)
                nc = nc + (v == _v(p)).astype(jnp.int32)
                plsc.addupdate_scatter(hist, [jnp.minimum(v, _v(99)), l16],
                                       ones_v, mask=fg)
                return fc, nc

            fc, nc = lax.fori_loop(0, 100, stats, (zero_v, zero_v))

            def smax_body(p, sm):
                return jnp.maximum(sm, hist[p, :])

            sm = lax.fori_loop(0, 100, smax_body, zero_v)

            off = g * 80 + c * 16
            fbuf[pl.ds(off, 16)] = fc
            nbuf[pl.ds(off, 16)] = nc
            sbuf[pl.ds(off, 16)] = sm
            return 0

        lax.fori_loop(0, 5, col_body, 0)
        return 0

    lax.fori_loop(0, GROUPS, group_body, 0)

    out0 = wid * IMGS_PER_TILE
    pltpu.sync_copy(fbuf, f_hbm.at[pl.ds(out0, IMGS_PER_TILE)])
    pltpu.sync_copy(nbuf, n_hbm.at[pl.ds(out0, IMGS_PER_TILE)])
    pltpu.sync_copy(sbuf, s_hbm.at[pl.ds(out0, IMGS_PER_TILE)])


def _sc_stats(x_flat):
    mesh = plsc.VectorSubcoreMesh(core_axis_name="c", subcore_axis_name="s")
    out = jax.ShapeDtypeStruct((N_IMGS,), jnp.int32)
    k = functools.partial(
        pl.kernel,
        out_type=[out, out, out],
        mesh=mesh,
        scratch_types=[
            pltpu.VMEM((24000,), jnp.float32),
            pltpu.VMEM((128, 16), jnp.int32),
            pltpu.VMEM((128, 16), jnp.int32),
            pltpu.VMEM((100, 16), jnp.int32),
            pltpu.VMEM((IMGS_PER_TILE,), jnp.int32),
            pltpu.VMEM((IMGS_PER_TILE,), jnp.int32),
            pltpu.VMEM((IMGS_PER_TILE,), jnp.int32),
        ],
        compiler_params=pltpu.CompilerParams(needs_layout_passes=False),
    )(_sc_stats_kernel)
    return k(x_flat)


def _lane_shift_right(x, sh):
    return jnp.concatenate(
        [jnp.zeros(x.shape[:-1] + (sh,), x.dtype), x[..., :-sh]], axis=-1)


def _assemble_kernel(f_ref, n_ref, s_ref, pc_ref, pq_ref, pm_ref, lac_ref,
                     fd_ref):
    f = f_ref[...].astype(jnp.float32)
    n = n_ref[...]
    s = s_ref[...].astype(jnp.float32)
    w = jnp.maximum(f, 1.0)
    inv100 = jnp.float32(0.01)
    fd = jnp.sum(1.0 / w, axis=1, keepdims=True) * inv100
    m1 = (jnp.sum(w, axis=1, keepdims=True) * inv100) ** 2
    m2 = jnp.sum(w * w, axis=1, keepdims=True) * inv100
    lac = (m2 - m1) / m1
    pq = jnp.sum((f >= 59.5).astype(jnp.float32), axis=1, keepdims=True) * inv100

    incl = n
    for sh in (1, 2, 4, 8, 16, 32, 64):
        incl = incl + _lane_shift_right(incl, sh)
    excl = incl - n
    row_tot = jnp.sum(n, axis=1, keepdims=True)
    ridx = lax.broadcasted_iota(jnp.int32, (640, 1), 0)
    rincl = row_tot
    for sh in (1, 2, 4, 8, 16):
        shifted = jnp.concatenate(
            [jnp.zeros((sh, 1), rincl.dtype), rincl[:-sh]], axis=0)
        rincl = rincl + jnp.where(ridx % 32 >= sh, shifted, 0)
    roff = rincl - row_tot
    off = (roff + excl + n).astype(jnp.float32)
    pc = jnp.sum(jnp.where(n > 0, off, 0.0), axis=1, keepdims=True) * inv100

    bg = jnp.float32(10000.0) - jnp.sum(f, axis=1, keepdims=True)
    pm = jnp.maximum(bg, jnp.max(s, axis=1, keepdims=True))

    pc_ref[...] = pc
    pq_ref[...] = pq
    pm_ref[...] = pm
    lac_ref[...] = lac
    fd_ref[...] = fd


def _assemble(f, n, s):
    outs = [jax.ShapeDtypeStruct((640, 1), jnp.float32)] * 5
    return pl.pallas_call(
        _assemble_kernel,
        out_shape=outs,
    )(f, n, s)


@jax.jit
def kernel(inputs):
    x_flat = inputs.reshape(-1)
    f, n, s = _sc_stats(x_flat)
    f2 = f.reshape(640, 100)
    n2 = n.reshape(640, 100)
    s2 = s.reshape(640, 100)
    pc, pq, pm, lac, fd = _assemble(f2, n2, s2)
    cols = [v.reshape(20, 32).T for v in (pc, pq, pm, lac, fd)]
    out = jnp.concatenate(cols, axis=1)
    return out.reshape(-1, 10, 10)


# speedup vs baseline: 23.4235x; 11.5616x over previous
import functools

import jax
import jax.numpy as jnp
from jax import lax
from jax.experimental import pallas as pl
from jax.experimental.pallas import tpu as pltpu
from jax.experimental.pallas import tpu_sc as plsc

SENT = 1000
N_IMGS = 64000
IMGS_PER_TILE = N_IMGS // 32
GROUPS = IMGS_PER_TILE // 80
BZ_BLK = 512


def _binarize_kernel(x_ref, c_ref, bits_ref, f_ref):
    x = x_ref[...]
    c = c_ref[...]
    lane = lax.broadcasted_iota(jnp.int32, x.shape, 1)
    ch = lane % 3
    cc = jnp.where(ch == 0, c[:, 0:1],
                   jnp.where(ch == 1, c[:, 1:2], c[:, 2:3]))
    ok = (jnp.abs(x - cc) <= jnp.float32(10.0)).astype(jnp.float32)
    sel = (lax.broadcasted_iota(jnp.int32, (300, 128), 0) // 3
           == lax.broadcasted_iota(jnp.int32, (300, 128), 1))
    s = lax.dot_general(ok, sel.astype(jnp.float32), (((1,), (0,)), ((), ())),
                        preferred_element_type=jnp.float32)
    bits = (s > 2.5).astype(jnp.int32)
    bits_ref[...] = bits
    f_ref[...] = jnp.sum(bits, axis=1, keepdims=True)


def _tc_binarize(x2, cen):
    grid = (N_IMGS // BZ_BLK,)
    return pl.pallas_call(
        _binarize_kernel,
        grid=grid,
        in_specs=[
            pl.BlockSpec((BZ_BLK, 300), lambda i: (i, 0)),
            pl.BlockSpec((BZ_BLK, 3), lambda i: (i, 0)),
        ],
        out_specs=[
            pl.BlockSpec((BZ_BLK, 128), lambda i: (i, 0)),
            pl.BlockSpec((BZ_BLK, 1), lambda i: (i, 0)),
        ],
        out_shape=[
            jax.ShapeDtypeStruct((N_IMGS, 128), jnp.int32),
            jax.ShapeDtypeStruct((N_IMGS, 1), jnp.int32),
        ],
    )(x2, cen)


def _v(x):
    return jnp.full((16,), x, jnp.int32)


def _sc_stats_kernel(bits_hbm, n_hbm, s_hbm, bbuf, lab, lab2, hist,
                     nbuf, sbuf):
    wid = lax.axis_index("s") * 2 + lax.axis_index("c")
    l16 = lax.iota(jnp.int32, 16)
    sent_v = _v(SENT)
    ones_v = jnp.ones((16,), jnp.int32)
    zero_v = jnp.zeros((16,), jnp.int32)

    for r in list(range(10)) + list(range(110, 128)):
        lab[r, :] = sent_v
        lab2[r, :] = sent_v

    def group_body(g, _):
        base = (wid * IMGS_PER_TILE + g * 80) * 128
        pltpu.sync_copy(bits_hbm.at[pl.ds(base, 80 * 128)], bbuf)

        def col_body(c, _):
            jb = (_v(c * 16) + l16) * _v(128)

            def binz(p, _):
                b = plsc.load_gather(bbuf, [jb + _v(p)])
                lab[p + 10, :] = jnp.where(b > zero_v, _v(p), sent_v)
                hist[p, :] = zero_v
                return 0

            lax.fori_loop(0, 100, binz, 0)

            def cc_cond(carry):
                return carry > 0

            def cc_body(carry):
                def pass1(p, _):
                    cur = lab[p + 10, :]
                    up = lab[p, :]
                    dn = lab[p + 20, :]
                    pv = _v(p)
                    lf = jnp.where(pv % _v(10) == zero_v, sent_v,
                                   lab[p + 9, :])
                    rt = jnp.where(pv % _v(10) == _v(9), sent_v,
                                   lab[p + 11, :])
                    m = jnp.minimum(jnp.minimum(cur, jnp.minimum(up, dn)),
                                    jnp.minimum(lf, rt))
                    lab2[p + 10, :] = jnp.where(cur != sent_v, m, sent_v)
                    return 0

                lax.fori_loop(0, 100, pass1, 0)

                def pass2(p, ch):
                    v = lab2[p + 10, :]
                    row = jnp.minimum(v, _v(100)) + _v(10)
                    gth = plsc.load_gather(lab2, [row, l16])
                    nv = jnp.where(v != sent_v, jnp.minimum(v, gth), sent_v)
                    old = lab[p + 10, :]
                    lab[p + 10, :] = nv
                    d = jnp.max((nv != old).astype(jnp.int32))
                    return jnp.maximum(ch, d)

                return lax.fori_loop(0, 100, pass2, 0)

            lax.while_loop(cc_cond, cc_body, jnp.int32(1))

            def stats(p, nc):
                v = lab[p + 10, :]
                fg = v != sent_v
                nc = nc + (v == _v(p)).astype(jnp.int32)
                plsc.addupdate_scatter(hist, [jnp.minimum(v, _v(99)), l16],
                                       ones_v, mask=fg)
                return nc

            nc = lax.fori_loop(0, 100, stats, zero_v)

            def smax_body(p, sm):
                return jnp.maximum(sm, hist[p, :])

            sm = lax.fori_loop(0, 100, smax_body, zero_v)

            off = g * 80 + c * 16
            nbuf[pl.ds(off, 16)] = nc
            sbuf[pl.ds(off, 16)] = sm
            return 0

        lax.fori_loop(0, 5, col_body, 0)
        return 0

    lax.fori_loop(0, GROUPS, group_body, 0)

    out0 = wid * IMGS_PER_TILE
    pltpu.sync_copy(nbuf, n_hbm.at[pl.ds(out0, IMGS_PER_TILE)])
    pltpu.sync_copy(sbuf, s_hbm.at[pl.ds(out0, IMGS_PER_TILE)])


def _sc_stats(bits_flat):
    mesh = plsc.VectorSubcoreMesh(core_axis_name="c", subcore_axis_name="s")
    out = jax.ShapeDtypeStruct((N_IMGS,), jnp.int32)
    k = functools.partial(
        pl.kernel,
        out_type=[out, out],
        mesh=mesh,
        scratch_types=[
            pltpu.VMEM((80 * 128,), jnp.int32),
            pltpu.VMEM((128, 16), jnp.int32),
            pltpu.VMEM((128, 16), jnp.int32),
            pltpu.VMEM((100, 16), jnp.int32),
            pltpu.VMEM((IMGS_PER_TILE,), jnp.int32),
            pltpu.VMEM((IMGS_PER_TILE,), jnp.int32),
        ],
        compiler_params=pltpu.CompilerParams(needs_layout_passes=False),
    )(_sc_stats_kernel)
    return k(bits_flat)


def _lane_shift_right(x, sh):
    return jnp.concatenate(
        [jnp.zeros(x.shape[:-1] + (sh,), x.dtype), x[..., :-sh]], axis=-1)


def _assemble_kernel(f_ref, n_ref, s_ref, pc_ref, pq_ref, pm_ref, lac_ref,
                     fd_ref):
    f = f_ref[...].astype(jnp.float32)
    n = n_ref[...]
    s = s_ref[...].astype(jnp.float32)
    w = jnp.maximum(f, 1.0)
    inv100 = jnp.float32(0.01)
    fd = jnp.sum(1.0 / w, axis=1, keepdims=True) * inv100
    m1 = (jnp.sum(w, axis=1, keepdims=True) * inv100) ** 2
    m2 = jnp.sum(w * w, axis=1, keepdims=True) * inv100
    lac = (m2 - m1) / m1
    pq = jnp.sum((f >= 59.5).astype(jnp.float32), axis=1, keepdims=True) * inv100

    incl = n
    for sh in (1, 2, 4, 8, 16, 32, 64):
        incl = incl + _lane_shift_right(incl, sh)
    excl = incl - n
    row_tot = jnp.sum(n, axis=1, keepdims=True)
    ridx = lax.broadcasted_iota(jnp.int32, (640, 1), 0)
    rincl = row_tot
    for sh in (1, 2, 4, 8, 16):
        shifted = jnp.concatenate(
            [jnp.zeros((sh, 1), rincl.dtype), rincl[:-sh]], axis=0)
        rincl = rincl + jnp.where(ridx % 32 >= sh, shifted, 0)
    roff = rincl - row_tot
    off = (roff + excl + n).astype(jnp.float32)
    pc = jnp.sum(jnp.where(n > 0, off, 0.0), axis=1, keepdims=True) * inv100

    bg = jnp.float32(10000.0) - jnp.sum(f, axis=1, keepdims=True)
    pm = jnp.maximum(bg, jnp.max(s, axis=1, keepdims=True))

    pc_ref[...] = pc
    pq_ref[...] = pq
    pm_ref[...] = pm
    lac_ref[...] = lac
    fd_ref[...] = fd


def _assemble(f, n, s):
    outs = [jax.ShapeDtypeStruct((640, 1), jnp.float32)] * 5
    return pl.pallas_call(
        _assemble_kernel,
        out_shape=outs,
    )(f, n, s)


@jax.jit
def kernel(inputs):
    x2 = inputs.reshape(N_IMGS, 300)
    cen = jnp.repeat(x2[4::10, 282:285], 10, axis=0)
    bits, f = _tc_binarize(x2, cen)
    n, s = _sc_stats(bits.reshape(-1))
    f2 = f.reshape(640, 100)
    n2 = n.reshape(640, 100)
    s2 = s.reshape(640, 100)
    pc, pq, pm, lac, fd = _assemble(f2, n2, s2)
    cols = [v.reshape(20, 32).T for v in (pc, pq, pm, lac, fd)]
    out = jnp.concatenate(cols, axis=1)
    return out.reshape(-1, 10, 10)


# speedup vs baseline: 80.7679x; 3.4482x over previous
import functools

import jax
import jax.numpy as jnp
from jax import lax
from jax.experimental import pallas as pl
from jax.experimental.pallas import tpu as pltpu
from jax.experimental.pallas import tpu_sc as plsc

SENT = 1000
N_IMGS = 64000
IMGS_PER_TILE = N_IMGS // 32
GROUPS = IMGS_PER_TILE // 80
BZ_BLK = 512


def _binarize_kernel(x_ref, c_ref, bits_ref, f_ref):
    x = x_ref[...]
    c = c_ref[...]
    lane = lax.broadcasted_iota(jnp.int32, x.shape, 1)
    ch = lane % 3
    cc = jnp.where(ch == 0, c[:, 0:1],
                   jnp.where(ch == 1, c[:, 1:2], c[:, 2:3]))
    ok = (jnp.abs(x - cc) <= jnp.float32(10.0)).astype(jnp.float32)
    sel = (lax.broadcasted_iota(jnp.int32, (300, 128), 0) // 3
           == lax.broadcasted_iota(jnp.int32, (300, 128), 1))
    s = lax.dot_general(ok, sel.astype(jnp.float32), (((1,), (0,)), ((), ())),
                        preferred_element_type=jnp.float32)
    bits = (s > 2.5).astype(jnp.int32)
    bits_ref[...] = bits
    f_ref[...] = jnp.sum(bits, axis=1, keepdims=True)


def _tc_binarize(x2, cen):
    grid = (N_IMGS // BZ_BLK,)
    return pl.pallas_call(
        _binarize_kernel,
        grid=grid,
        in_specs=[
            pl.BlockSpec((BZ_BLK, 300), lambda i: (i, 0)),
            pl.BlockSpec((BZ_BLK, 3), lambda i: (i, 0)),
        ],
        out_specs=[
            pl.BlockSpec((BZ_BLK, 128), lambda i: (i, 0)),
            pl.BlockSpec((BZ_BLK, 1), lambda i: (i, 0)),
        ],
        out_shape=[
            jax.ShapeDtypeStruct((N_IMGS, 128), jnp.int32),
            jax.ShapeDtypeStruct((N_IMGS, 1), jnp.int32),
        ],
    )(x2, cen)


def _v(x):
    return jnp.full((16,), x, jnp.int32)


def _sc_stats_kernel(bits_hbm, n_hbm, s_hbm, bbuf, lab, lab2, hist,
                     nbuf, sbuf):
    wid = lax.axis_index("s") * 2 + lax.axis_index("c")
    l16 = lax.iota(jnp.int32, 16)
    sent_v = _v(SENT)
    ones_v = jnp.ones((16,), jnp.int32)
    zero_v = jnp.zeros((16,), jnp.int32)

    for r in list(range(10)) + list(range(110, 128)):
        lab[r, :] = sent_v
        lab2[r, :] = sent_v

    hundred_v = jnp.full((16,), 100, jnp.int32)

    def group_body(g, _):
        base = (wid * IMGS_PER_TILE + g * 80) * 128
        pltpu.sync_copy(bits_hbm.at[pl.ds(base, 80 * 128)], bbuf)

        @plsc.parallel_loop(0, 640, unroll=8, carry=zero_v)
        def total_v(i, acc):
            return acc + bbuf[pl.ds(i * 16, 16)]

        total = jnp.sum(total_v)

        @pl.when(total == 8000)
        def _():
            for c in range(5):
                off = g * 80 + c * 16
                nbuf[pl.ds(off, 16)] = ones_v
                sbuf[pl.ds(off, 16)] = hundred_v

        @pl.when(total != 8000)
        def _():
            lax.fori_loop(0, 5, col_body_fn(g), 0)

        return 0

    def col_body_fn(g):
        def col_body(c, _):
            jb = (_v(c * 16) + l16) * _v(128)

            def binz(p, fc):
                b = plsc.load_gather(bbuf, [jb + _v(p)])
                lab[p + 10, :] = jnp.where(b > zero_v, _v(p), sent_v)
                hist[p, :] = zero_v
                return fc + (b > zero_v).astype(jnp.int32)

            fcnt = lax.fori_loop(0, 100, binz, zero_v)
            off = g * 80 + c * 16

            @pl.when(jnp.min(fcnt) == 100)
            def _():
                nbuf[pl.ds(off, 16)] = ones_v
                sbuf[pl.ds(off, 16)] = hundred_v

            @pl.when(jnp.min(fcnt) != 100)
            def _():
                _cc_and_stats(off)
            return 0

        return col_body

    def _cc_and_stats(off):
            def cc_cond(carry):
                return carry > 0

            def cc_body(carry):
                def pass1(p, _):
                    cur = lab[p + 10, :]
                    up = lab[p, :]
                    dn = lab[p + 20, :]
                    pv = _v(p)
                    lf = jnp.where(pv % _v(10) == zero_v, sent_v,
                                   lab[p + 9, :])
                    rt = jnp.where(pv % _v(10) == _v(9), sent_v,
                                   lab[p + 11, :])
                    m = jnp.minimum(jnp.minimum(cur, jnp.minimum(up, dn)),
                                    jnp.minimum(lf, rt))
                    lab2[p + 10, :] = jnp.where(cur != sent_v, m, sent_v)
                    return 0

                lax.fori_loop(0, 100, pass1, 0)

                def pass2(p, ch):
                    v = lab2[p + 10, :]
                    row = jnp.minimum(v, _v(100)) + _v(10)
                    gth = plsc.load_gather(lab2, [row, l16])
                    nv = jnp.where(v != sent_v, jnp.minimum(v, gth), sent_v)
                    old = lab[p + 10, :]
                    lab[p + 10, :] = nv
                    d = jnp.max((nv != old).astype(jnp.int32))
                    return jnp.maximum(ch, d)

                return lax.fori_loop(0, 100, pass2, 0)

            lax.while_loop(cc_cond, cc_body, jnp.int32(1))

            def stats(p, nc):
                v = lab[p + 10, :]
                fg = v != sent_v
                nc = nc + (v == _v(p)).astype(jnp.int32)
                plsc.addupdate_scatter(hist, [jnp.minimum(v, _v(99)), l16],
                                       ones_v, mask=fg)
                return nc

            nc = lax.fori_loop(0, 100, stats, zero_v)

            def smax_body(p, sm):
                return jnp.maximum(sm, hist[p, :])

            sm = lax.fori_loop(0, 100, smax_body, zero_v)

            nbuf[pl.ds(off, 16)] = nc
            sbuf[pl.ds(off, 16)] = sm

    lax.fori_loop(0, GROUPS, group_body, 0)

    out0 = wid * IMGS_PER_TILE
    pltpu.sync_copy(nbuf, n_hbm.at[pl.ds(out0, IMGS_PER_TILE)])
    pltpu.sync_copy(sbuf, s_hbm.at[pl.ds(out0, IMGS_PER_TILE)])


def _sc_stats(bits_flat):
    mesh = plsc.VectorSubcoreMesh(core_axis_name="c", subcore_axis_name="s")
    out = jax.ShapeDtypeStruct((N_IMGS,), jnp.int32)
    k = functools.partial(
        pl.kernel,
        out_type=[out, out],
        mesh=mesh,
        scratch_types=[
            pltpu.VMEM((80 * 128,), jnp.int32),
            pltpu.VMEM((128, 16), jnp.int32),
            pltpu.VMEM((128, 16), jnp.int32),
            pltpu.VMEM((100, 16), jnp.int32),
            pltpu.VMEM((IMGS_PER_TILE,), jnp.int32),
            pltpu.VMEM((IMGS_PER_TILE,), jnp.int32),
        ],
        compiler_params=pltpu.CompilerParams(needs_layout_passes=False),
    )(_sc_stats_kernel)
    return k(bits_flat)


def _lane_shift_right(x, sh):
    return jnp.concatenate(
        [jnp.zeros(x.shape[:-1] + (sh,), x.dtype), x[..., :-sh]], axis=-1)


def _assemble_kernel(f_ref, n_ref, s_ref, pc_ref, pq_ref, pm_ref, lac_ref,
                     fd_ref):
    f = f_ref[...].astype(jnp.float32)
    n = n_ref[...]
    s = s_ref[...].astype(jnp.float32)
    w = jnp.maximum(f, 1.0)
    inv100 = jnp.float32(0.01)
    fd = jnp.sum(1.0 / w, axis=1, keepdims=True) * inv100
    m1 = (jnp.sum(w, axis=1, keepdims=True) * inv100) ** 2
    m2 = jnp.sum(w * w, axis=1, keepdims=True) * inv100
    lac = (m2 - m1) / m1
    pq = jnp.sum((f >= 59.5).astype(jnp.float32), axis=1, keepdims=True) * inv100

    incl = n
    for sh in (1, 2, 4, 8, 16, 32, 64):
        incl = incl + _lane_shift_right(incl, sh)
    excl = incl - n
    row_tot = jnp.sum(n, axis=1, keepdims=True)
    ridx = lax.broadcasted_iota(jnp.int32, (640, 1), 0)
    rincl = row_tot
    for sh in (1, 2, 4, 8, 16):
        shifted = jnp.concatenate(
            [jnp.zeros((sh, 1), rincl.dtype), rincl[:-sh]], axis=0)
        rincl = rincl + jnp.where(ridx % 32 >= sh, shifted, 0)
    roff = rincl - row_tot
    off = (roff + excl + n).astype(jnp.float32)
    pc = jnp.sum(jnp.where(n > 0, off, 0.0), axis=1, keepdims=True) * inv100

    bg = jnp.float32(10000.0) - jnp.sum(f, axis=1, keepdims=True)
    pm = jnp.maximum(bg, jnp.max(s, axis=1, keepdims=True))

    pc_ref[...] = pc
    pq_ref[...] = pq
    pm_ref[...] = pm
    lac_ref[...] = lac
    fd_ref[...] = fd


def _assemble(f, n, s):
    outs = [jax.ShapeDtypeStruct((640, 1), jnp.float32)] * 5
    return pl.pallas_call(
        _assemble_kernel,
        out_shape=outs,
    )(f, n, s)


@jax.jit
def kernel(inputs):
    x2 = inputs.reshape(N_IMGS, 300)
    cen = jnp.repeat(x2[4::10, 282:285], 10, axis=0)
    bits, f = _tc_binarize(x2, cen)
    n, s = _sc_stats(bits.reshape(-1))
    f2 = f.reshape(640, 100)
    n2 = n.reshape(640, 100)
    s2 = s.reshape(640, 100)
    pc, pq, pm, lac, fd = _assemble(f2, n2, s2)
    cols = [v.reshape(20, 32).T for v in (pc, pq, pm, lac, fd)]
    out = jnp.concatenate(cols, axis=1)
    return out.reshape(-1, 10, 10)


# speedup vs baseline: 91.1942x; 1.1291x over previous
import functools

import jax
import jax.numpy as jnp
from jax import lax
from jax.experimental import pallas as pl
from jax.experimental.pallas import tpu as pltpu
from jax.experimental.pallas import tpu_sc as plsc

SENT = 1000
N_IMGS = 64000
IMGS_PER_TILE = N_IMGS // 32
GROUPS = IMGS_PER_TILE // 80
BZ_BLK = 1000


def _sub_shift(x, d):
    if d > 0:
        pad = jnp.zeros((d,) + x.shape[1:], x.dtype)
        return jnp.concatenate([x[d:], pad], axis=0)
    if d < 0:
        pad = jnp.zeros((-d,) + x.shape[1:], x.dtype)
        return jnp.concatenate([pad, x[:d]], axis=0)
    return x


def _binarize_kernel(x_ref, bits_ref):
    x = x_ref[...]
    xc = x[:, 282:285]
    rmod = lax.broadcasted_iota(jnp.int32, (BZ_BLK, 3), 0) % 10
    c3 = jnp.zeros((BZ_BLK, 3), jnp.float32)
    for r in range(10):
        c3 = jnp.where(rmod == r, _sub_shift(xc, 4 - r), c3)

    lane = lax.broadcasted_iota(jnp.int32, x.shape, 1)
    ch = lane % 3
    cc = jnp.where(ch == 0, c3[:, 0:1],
                   jnp.where(ch == 1, c3[:, 1:2], c3[:, 2:3]))
    ok = (jnp.abs(x - cc) <= jnp.float32(10.0)).astype(jnp.float32)
    sel3 = (lax.broadcasted_iota(jnp.int32, (300, 128), 0) // 3
            == lax.broadcasted_iota(jnp.int32, (300, 128), 1))
    sb = lax.dot_general(ok, sel3.astype(jnp.float32),
                         (((1,), (0,)), ((), ())),
                         preferred_element_type=jnp.float32)
    bits_ref[...] = (sb > 2.5).astype(jnp.int32)


def _tc_binarize(x2):
    grid = (N_IMGS // BZ_BLK,)
    return pl.pallas_call(
        _binarize_kernel,
        grid=grid,
        in_specs=[pl.BlockSpec((BZ_BLK, 300), lambda i: (i, 0))],
        out_specs=[pl.BlockSpec((BZ_BLK, 128), lambda i: (i, 0))],
        out_shape=[jax.ShapeDtypeStruct((N_IMGS, 128), jnp.int32)],
    )(x2)


def _v(x):
    return jnp.full((16,), x, jnp.int32)


def _sc_stats_kernel(bits_hbm, f_hbm, n_hbm, s_hbm, bbuf, lab, lab2, hist,
                     fbuf, nbuf, sbuf):
    wid = lax.axis_index("s") * 2 + lax.axis_index("c")
    l16 = lax.iota(jnp.int32, 16)
    sent_v = _v(SENT)
    ones_v = jnp.ones((16,), jnp.int32)
    zero_v = jnp.zeros((16,), jnp.int32)
    hundred_v = jnp.full((16,), 100, jnp.int32)

    for r in list(range(10)) + list(range(110, 128)):
        lab[r, :] = sent_v
        lab2[r, :] = sent_v

    def zinit(i, _):
        fbuf[pl.ds(i * 16, 16)] = zero_v
        nbuf[pl.ds(i * 16, 16)] = zero_v
        sbuf[pl.ds(i * 16, 16)] = zero_v
        return 0

    lax.fori_loop(0, 160, zinit, 0)

    def store16(buf, off, val):
        li = _v(off) + l16
        plsc.store_scatter(buf, [(li // hundred_v) * _v(128)
                                 + li % hundred_v], val)

    def group_body(g, _):
        img0 = pl.multiple_of(wid * IMGS_PER_TILE + g * 80, 8)
        pltpu.sync_copy(bits_hbm.at[pl.ds(img0, 80), :], bbuf)

        @plsc.parallel_loop(0, 640, unroll=8, carry=zero_v)
        def total_v(i, acc):
            return acc + bbuf[i // 8, pl.ds((i % 8) * 16, 16)]

        total = jnp.sum(total_v)

        @pl.when(total == 8000)
        def _():
            for c in range(5):
                store16(fbuf, g * 80 + c * 16, hundred_v)
                store16(nbuf, g * 80 + c * 16, ones_v)
                store16(sbuf, g * 80 + c * 16, hundred_v)

        @pl.when(total != 8000)
        def _():
            lax.fori_loop(0, 5, col_body_fn(g), 0)

        return 0

    def col_body_fn(g):
        def col_body(c, _):
            jr = _v(c * 16) + l16

            def binz(p, fc):
                b = plsc.load_gather(bbuf, [jr, _v(p)])
                lab[p + 10, :] = jnp.where(b > zero_v, _v(p), sent_v)
                hist[p, :] = zero_v
                return fc + (b > zero_v).astype(jnp.int32)

            fcnt = lax.fori_loop(0, 100, binz, zero_v)
            off = g * 80 + c * 16
            store16(fbuf, off, fcnt)

            @pl.when(jnp.min(fcnt) == 100)
            def _():
                store16(nbuf, off, ones_v)
                store16(sbuf, off, hundred_v)

            @pl.when(jnp.min(fcnt) != 100)
            def _():
                _cc_and_stats(off)
            return 0

        return col_body

    def _cc_and_stats(off):
            def cc_cond(carry):
                return carry > 0

            def cc_body(carry):
                def pass1(p, _):
                    cur = lab[p + 10, :]
                    up = lab[p, :]
                    dn = lab[p + 20, :]
                    pv = _v(p)
                    lf = jnp.where(pv % _v(10) == zero_v, sent_v,
                                   lab[p + 9, :])
                    rt = jnp.where(pv % _v(10) == _v(9), sent_v,
                                   lab[p + 11, :])
                    m = jnp.minimum(jnp.minimum(cur, jnp.minimum(up, dn)),
                                    jnp.minimum(lf, rt))
                    lab2[p + 10, :] = jnp.where(cur != sent_v, m, sent_v)
                    return 0

                lax.fori_loop(0, 100, pass1, 0)

                def pass2(p, ch):
                    v = lab2[p + 10, :]
                    row = jnp.minimum(v, _v(100)) + _v(10)
                    gth = plsc.load_gather(lab2, [row, l16])
                    nv = jnp.where(v != sent_v, jnp.minimum(v, gth), sent_v)
                    old = lab[p + 10, :]
                    lab[p + 10, :] = nv
                    d = jnp.max((nv != old).astype(jnp.int32))
                    return jnp.maximum(ch, d)

                return lax.fori_loop(0, 100, pass2, 0)

            lax.while_loop(cc_cond, cc_body, jnp.int32(1))

            def stats(p, nc):
                v = lab[p + 10, :]
                fg = v != sent_v
                nc = nc + (v == _v(p)).astype(jnp.int32)
                plsc.addupdate_scatter(hist, [jnp.minimum(v, _v(99)), l16],
                                       ones_v, mask=fg)
                return nc

            nc = lax.fori_loop(0, 100, stats, zero_v)

            def smax_body(p, sm):
                return jnp.maximum(sm, hist[p, :])

            sm = lax.fori_loop(0, 100, smax_body, zero_v)

            store16(nbuf, off, nc)
            store16(sbuf, off, sm)

    lax.fori_loop(0, GROUPS, group_body, 0)

    base = pl.multiple_of(wid * 2560, 8)
    pltpu.sync_copy(fbuf, f_hbm.at[pl.ds(base, 2560)])
    pltpu.sync_copy(nbuf, n_hbm.at[pl.ds(base, 2560)])
    pltpu.sync_copy(sbuf, s_hbm.at[pl.ds(base, 2560)])


def _sc_stats(bits):
    mesh = plsc.VectorSubcoreMesh(core_axis_name="c", subcore_axis_name="s")
    out = jax.ShapeDtypeStruct((640 * 128,), jnp.int32)
    k = functools.partial(
        pl.kernel,
        out_type=[out, out, out],
        mesh=mesh,
        scratch_types=[
            pltpu.VMEM((80, 128), jnp.int32),
            pltpu.VMEM((128, 16), jnp.int32),
            pltpu.VMEM((128, 16), jnp.int32),
            pltpu.VMEM((100, 16), jnp.int32),
            pltpu.VMEM((2560,), jnp.int32),
            pltpu.VMEM((2560,), jnp.int32),
            pltpu.VMEM((2560,), jnp.int32),
        ],
        compiler_params=pltpu.CompilerParams(needs_layout_passes=False),
    )(_sc_stats_kernel)
    return k(bits)


def _lane_shift_right(x, sh):
    return jnp.concatenate(
        [jnp.zeros(x.shape[:-1] + (sh,), x.dtype), x[..., :-sh]], axis=-1)


def _assemble_kernel(f_ref, n_ref, s_ref, pc_ref, pq_ref, pm_ref, lac_ref,
                     fd_ref):
    f = f_ref[...].astype(jnp.float32)
    n = n_ref[...]
    s = s_ref[...].astype(jnp.float32)
    lmask = lax.broadcasted_iota(jnp.int32, f.shape, 1) < 100
    w = jnp.maximum(f, 1.0)
    inv100 = jnp.float32(0.01)
    zero = jnp.float32(0.0)
    fd = jnp.sum(jnp.where(lmask, 1.0 / w, zero), axis=1,
                 keepdims=True) * inv100
    wm = jnp.where(lmask, w, zero)
    m1 = (jnp.sum(wm, axis=1, keepdims=True) * inv100) ** 2
    m2 = jnp.sum(wm * wm, axis=1, keepdims=True) * inv100
    lac = (m2 - m1) / m1
    pq = jnp.sum((f >= 59.5).astype(jnp.float32), axis=1,
                 keepdims=True) * inv100
    bg = jnp.float32(10000.0) - jnp.sum(f, axis=1, keepdims=True)

    incl = n
    for sh in (1, 2, 4, 8, 16, 32, 64):
        incl = incl + _lane_shift_right(incl, sh)
    excl = incl - n
    row_tot = jnp.sum(n, axis=1, keepdims=True)
    ridx = lax.broadcasted_iota(jnp.int32, (640, 1), 0)
    rincl = row_tot
    for sh in (1, 2, 4, 8, 16):
        shifted = jnp.concatenate(
            [jnp.zeros((sh, 1), rincl.dtype), rincl[:-sh]], axis=0)
        rincl = rincl + jnp.where(ridx % 32 >= sh, shifted, 0)
    roff = rincl - row_tot
    off = (roff + excl + n).astype(jnp.float32)
    pc = jnp.sum(jnp.where(n > 0, off, 0.0), axis=1, keepdims=True) * inv100

    pm = jnp.maximum(bg, jnp.max(s, axis=1, keepdims=True))

    pc_ref[...] = pc
    pq_ref[...] = pq
    pm_ref[...] = pm
    lac_ref[...] = lac
    fd_ref[...] = fd


def _assemble(f, n, s):
    outs = [jax.ShapeDtypeStruct((640, 1), jnp.float32)] * 5
    return pl.pallas_call(
        _assemble_kernel,
        out_shape=outs,
    )(f, n, s)


@jax.jit
def kernel(inputs):
    x2 = inputs.reshape(N_IMGS, 300)
    bits, = _tc_binarize(x2)
    f, n, s = _sc_stats(bits)
    pc, pq, pm, lac, fd = _assemble(f.reshape(640, 128), n.reshape(640, 128),
                                    s.reshape(640, 128))
    cols = [v.reshape(20, 32).T for v in (pc, pq, pm, lac, fd)]
    out = jnp.concatenate(cols, axis=1)
    return out.reshape(-1, 10, 10)


# speedup vs baseline: 92.0643x; 1.0095x over previous
import functools

import jax
import jax.numpy as jnp
from jax import lax
from jax.experimental import pallas as pl
from jax.experimental.pallas import tpu as pltpu
from jax.experimental.pallas import tpu_sc as plsc

SENT = 1000
N_IMGS = 64000
IMGS_PER_TILE = N_IMGS // 32
GROUPS = IMGS_PER_TILE // 80
BZ_BLK = 1000


def _sub_shift(x, d):
    if d > 0:
        pad = jnp.zeros((d,) + x.shape[1:], x.dtype)
        return jnp.concatenate([x[d:], pad], axis=0)
    if d < 0:
        pad = jnp.zeros((-d,) + x.shape[1:], x.dtype)
        return jnp.concatenate([pad, x[:d]], axis=0)
    return x


def _binarize_kernel(x_ref, bits_ref):
    x = x_ref[...]
    xc = x[:, 282:285]
    rmod = lax.broadcasted_iota(jnp.int32, (BZ_BLK, 3), 0) % 10
    c3 = jnp.zeros((BZ_BLK, 3), jnp.float32)
    for r in range(10):
        c3 = jnp.where(rmod == r, _sub_shift(xc, 4 - r), c3)

    lane = lax.broadcasted_iota(jnp.int32, x.shape, 1)
    ch = lane % 3
    cc = jnp.where(ch == 0, c3[:, 0:1],
                   jnp.where(ch == 1, c3[:, 1:2], c3[:, 2:3]))
    ok = (jnp.abs(x - cc) <= jnp.float32(10.0)).astype(jnp.bfloat16)
    sel3 = (lax.broadcasted_iota(jnp.int32, (300, 128), 0) // 3
            == lax.broadcasted_iota(jnp.int32, (300, 128), 1))
    sb = lax.dot_general(ok, sel3.astype(jnp.bfloat16),
                         (((1,), (0,)), ((), ())),
                         preferred_element_type=jnp.float32)
    bits_ref[...] = (sb > 2.5).astype(jnp.int32)


def _tc_binarize(x2):
    grid = (N_IMGS // BZ_BLK,)
    return pl.pallas_call(
        _binarize_kernel,
        grid=grid,
        in_specs=[pl.BlockSpec((BZ_BLK, 300), lambda i: (i, 0))],
        out_specs=[pl.BlockSpec((BZ_BLK, 128), lambda i: (i, 0))],
        out_shape=[jax.ShapeDtypeStruct((N_IMGS, 128), jnp.int32)],
    )(x2)


def _v(x):
    return jnp.full((16,), x, jnp.int32)


def _sc_stats_kernel(bits_hbm, f_hbm, n_hbm, s_hbm, bbuf, lab, lab2, hist,
                     fbuf, nbuf, sbuf):
    wid = lax.axis_index("s") * 2 + lax.axis_index("c")
    l16 = lax.iota(jnp.int32, 16)
    sent_v = _v(SENT)
    ones_v = jnp.ones((16,), jnp.int32)
    zero_v = jnp.zeros((16,), jnp.int32)
    hundred_v = jnp.full((16,), 100, jnp.int32)

    for r in list(range(10)) + list(range(110, 128)):
        lab[r, :] = sent_v
        lab2[r, :] = sent_v

    def zinit(i, _):
        fbuf[pl.ds(i * 16, 16)] = zero_v
        nbuf[pl.ds(i * 16, 16)] = zero_v
        sbuf[pl.ds(i * 16, 16)] = zero_v
        return 0

    lax.fori_loop(0, 160, zinit, 0)

    def store16(buf, off, val):
        li = _v(off) + l16
        plsc.store_scatter(buf, [(li // hundred_v) * _v(128)
                                 + li % hundred_v], val)

    def group_body(g, _):
        img0 = pl.multiple_of(wid * IMGS_PER_TILE + g * 80, 8)
        pltpu.sync_copy(bits_hbm.at[pl.ds(img0, 80), :], bbuf)

        @plsc.parallel_loop(0, 640, unroll=8, carry=zero_v)
        def total_v(i, acc):
            return acc + bbuf[i // 8, pl.ds((i % 8) * 16, 16)]

        total = jnp.sum(total_v)

        @pl.when(total == 8000)
        def _():
            for c in range(5):
                store16(fbuf, g * 80 + c * 16, hundred_v)
                store16(nbuf, g * 80 + c * 16, ones_v)
                store16(sbuf, g * 80 + c * 16, hundred_v)

        @pl.when(total != 8000)
        def _():
            lax.fori_loop(0, 5, col_body_fn(g), 0)

        return 0

    def col_body_fn(g):
        def col_body(c, _):
            jr = _v(c * 16) + l16

            def binz(p, fc):
                b = plsc.load_gather(bbuf, [jr, _v(p)])
                lab[p + 10, :] = jnp.where(b > zero_v, _v(p), sent_v)
                hist[p, :] = zero_v
                return fc + (b > zero_v).astype(jnp.int32)

            fcnt = lax.fori_loop(0, 100, binz, zero_v)
            off = g * 80 + c * 16
            store16(fbuf, off, fcnt)

            @pl.when(jnp.min(fcnt) == 100)
            def _():
                store16(nbuf, off, ones_v)
                store16(sbuf, off, hundred_v)

            @pl.when(jnp.min(fcnt) != 100)
            def _():
                _cc_and_stats(off)
            return 0

        return col_body

    def _cc_and_stats(off):
            def cc_cond(carry):
                return carry > 0

            def cc_body(carry):
                def pass1(p, _):
                    cur = lab[p + 10, :]
                    up = lab[p, :]
                    dn = lab[p + 20, :]
                    pv = _v(p)
                    lf = jnp.where(pv % _v(10) == zero_v, sent_v,
                                   lab[p + 9, :])
                    rt = jnp.where(pv % _v(10) == _v(9), sent_v,
                                   lab[p + 11, :])
                    m = jnp.minimum(jnp.minimum(cur, jnp.minimum(up, dn)),
                                    jnp.minimum(lf, rt))
                    lab2[p + 10, :] = jnp.where(cur != sent_v, m, sent_v)
                    return 0

                lax.fori_loop(0, 100, pass1, 0)

                def pass2(p, ch):
                    v = lab2[p + 10, :]
                    row = jnp.minimum(v, _v(100)) + _v(10)
                    gth = plsc.load_gather(lab2, [row, l16])
                    nv = jnp.where(v != sent_v, jnp.minimum(v, gth), sent_v)
                    old = lab[p + 10, :]
                    lab[p + 10, :] = nv
                    d = jnp.max((nv != old).astype(jnp.int32))
                    return jnp.maximum(ch, d)

                return lax.fori_loop(0, 100, pass2, 0)

            lax.while_loop(cc_cond, cc_body, jnp.int32(1))

            def stats(p, nc):
                v = lab[p + 10, :]
                fg = v != sent_v
                nc = nc + (v == _v(p)).astype(jnp.int32)
                plsc.addupdate_scatter(hist, [jnp.minimum(v, _v(99)), l16],
                                       ones_v, mask=fg)
                return nc

            nc = lax.fori_loop(0, 100, stats, zero_v)

            def smax_body(p, sm):
                return jnp.maximum(sm, hist[p, :])

            sm = lax.fori_loop(0, 100, smax_body, zero_v)

            store16(nbuf, off, nc)
            store16(sbuf, off, sm)

    lax.fori_loop(0, GROUPS, group_body, 0)

    base = pl.multiple_of(wid * 2560, 8)
    pltpu.sync_copy(fbuf, f_hbm.at[pl.ds(base, 2560)])
    pltpu.sync_copy(nbuf, n_hbm.at[pl.ds(base, 2560)])
    pltpu.sync_copy(sbuf, s_hbm.at[pl.ds(base, 2560)])


def _sc_stats(bits):
    mesh = plsc.VectorSubcoreMesh(core_axis_name="c", subcore_axis_name="s")
    out = jax.ShapeDtypeStruct((640 * 128,), jnp.int32)
    k = functools.partial(
        pl.kernel,
        out_type=[out, out, out],
        mesh=mesh,
        scratch_types=[
            pltpu.VMEM((80, 128), jnp.int32),
            pltpu.VMEM((128, 16), jnp.int32),
            pltpu.VMEM((128, 16), jnp.int32),
            pltpu.VMEM((100, 16), jnp.int32),
            pltpu.VMEM((2560,), jnp.int32),
            pltpu.VMEM((2560,), jnp.int32),
            pltpu.VMEM((2560,), jnp.int32),
        ],
        compiler_params=pltpu.CompilerParams(needs_layout_passes=False),
    )(_sc_stats_kernel)
    return k(bits)


def _lane_shift_right(x, sh):
    return jnp.concatenate(
        [jnp.zeros(x.shape[:-1] + (sh,), x.dtype), x[..., :-sh]], axis=-1)


def _assemble_kernel(f_ref, n_ref, s_ref, out_ref):
    f = f_ref[...].astype(jnp.float32)
    n = n_ref[...]
    s = s_ref[...].astype(jnp.float32)
    lmask = lax.broadcasted_iota(jnp.int32, f.shape, 1) < 100
    w = jnp.maximum(f, 1.0)
    inv100 = jnp.float32(0.01)
    zero = jnp.float32(0.0)
    fd = jnp.sum(jnp.where(lmask, 1.0 / w, zero), axis=1,
                 keepdims=True) * inv100
    wm = jnp.where(lmask, w, zero)
    m1 = (jnp.sum(wm, axis=1, keepdims=True) * inv100) ** 2
    m2 = jnp.sum(wm * wm, axis=1, keepdims=True) * inv100
    lac = (m2 - m1) / m1
    pq = jnp.sum((f >= 59.5).astype(jnp.float32), axis=1,
                 keepdims=True) * inv100
    bg = jnp.float32(10000.0) - jnp.sum(f, axis=1, keepdims=True)

    incl = n
    for sh in (1, 2, 4, 8, 16, 32, 64):
        incl = incl + _lane_shift_right(incl, sh)
    excl = incl - n
    row_tot = jnp.sum(n, axis=1, keepdims=True)
    ridx = lax.broadcasted_iota(jnp.int32, (640, 1), 0)
    rincl = row_tot
    for sh in (1, 2, 4, 8, 16):
        shifted = jnp.concatenate(
            [jnp.zeros((sh, 1), rincl.dtype), rincl[:-sh]], axis=0)
        rincl = rincl + jnp.where(ridx % 32 >= sh, shifted, 0)
    roff = rincl - row_tot
    off = (roff + excl + n).astype(jnp.float32)
    pc = jnp.sum(jnp.where(n > 0, off, 0.0), axis=1, keepdims=True) * inv100

    pm = jnp.maximum(bg, jnp.max(s, axis=1, keepdims=True))

    out_ref[...] = jnp.concatenate(
        [pc, pq, pm, lac, fd, jnp.zeros((640, 3), jnp.float32)], axis=1)


def _assemble(f, n, s):
    return pl.pallas_call(
        _assemble_kernel,
        out_shape=jax.ShapeDtypeStruct((640, 8), jnp.float32),
    )(f, n, s)


@jax.jit
def kernel(inputs):
    x2 = inputs.reshape(N_IMGS, 300)
    bits, = _tc_binarize(x2)
    f, n, s = _sc_stats(bits)
    feats = _assemble(f.reshape(640, 128), n.reshape(640, 128),
                      s.reshape(640, 128))
    out = feats.reshape(20, 32, 8).transpose(1, 2, 0)[:, :5, :]
    return out.reshape(-1, 10, 10)
